# baseline (speedup 1.0000x reference)
import jax
import jax.numpy as jnp
from jax import lax
from jax.experimental import pallas as pl
from jax.experimental.pallas import tpu as pltpu

_N = 8192
_D = 128
_L = 100
_C = 64
_NCHUNK = 2

def _a_stage(x, k, j, W):
    g = _N // (2 * j)
    xr = x.reshape(g, 2, j, W)
    a = xr[:, 0]
    b = xr[:, 1]
    giota = lax.broadcasted_iota(jnp.int32, (g, 1, 1), 0) * (2 * j)
    gasc = (giota & k) == 0
    lo = jnp.minimum(a, b)
    hi = jnp.maximum(a, b)
    first = jnp.where(gasc, lo, hi)
    second = jnp.where(gasc, hi, lo)
    return jnp.stack([first, second], axis=1).reshape(_N, W)


def _b_stage(blocks, k, j):
    rows = _N // 8
    rasc = None
    if k >= 8:
        riota = lax.broadcasted_iota(jnp.int32, (rows, 1), 0)
        rasc = (riota & (k // 8)) == 0
    out = []
    for t in range(8):
        a = blocks[t]
        b = blocks[t ^ j]
        lo = jnp.minimum(a, b)
        hi = jnp.maximum(a, b)
        bc = (t & j) == 0
        if k < 8:
            take_lo = ((t & k) == 0) == bc
            out.append(lo if take_lo else hi)
        else:
            out.append(jnp.where(rasc == bc, lo, hi))
    return out


def _to_blocks(xa, W):
    xb = xa.reshape(_N // 8, 8 * W)
    return [xb[:, t * W:(t + 1) * W] for t in range(8)]


def _from_blocks(blocks, W):
    return jnp.concatenate(blocks, axis=1).reshape(_N, W)


def _sort_cols(x):
    W = x.shape[1]
    blocks = _to_blocks(x, W)
    for k in (2, 4, 8):
        j = k // 2
        while j >= 1:
            blocks = _b_stage(blocks, k, j)
            j //= 2
    k = 16
    while k <= _N:
        xa = _from_blocks(blocks, W)
        j = k // 2
        while j >= 8:
            xa = _a_stage(xa, k, j, W)
            j //= 2
        blocks = _to_blocks(xa, W)
        for j in (4, 2, 1):
            blocks = _b_stage(blocks, k, j)
        k *= 2
    return _from_blocks(blocks, W)


def _swd_kernel(p_ref, q_ref, proj_ref, out_ref):
    P = p_ref[0]
    Q = q_ref[0]
    proj = proj_ref[0]
    Xp = jnp.dot(P, proj, preferred_element_type=jnp.float32)
    Yp = jnp.dot(Q, proj, preferred_element_type=jnp.float32)
    x = _sort_cols(jnp.concatenate([Xp, Yp], axis=1))
    d = x[:, :_C] - x[:, _C:]
    m = jnp.mean(d * d, axis=0, keepdims=True)
    out_ref[0, 0] = jnp.concatenate([m, jnp.zeros((1, _C), jnp.float32)], axis=1)


def kernel(P_batch, Q_batch, projections):
    B = P_batch.shape[0]
    projp = jnp.zeros((_D, _NCHUNK * _C), jnp.float32).at[:, :_L].set(projections)
    projc = projp.reshape(_D, _NCHUNK, _C).transpose(1, 0, 2)
    wpp = pl.pallas_call(
        _swd_kernel,
        grid=(B, _NCHUNK),
        in_specs=[
            pl.BlockSpec((1, _N, _D), lambda b, c: (b, 0, 0)),
            pl.BlockSpec((1, _N, _D), lambda b, c: (b, 0, 0)),
            pl.BlockSpec((1, _D, _C), lambda b, c: (c, 0, 0)),
        ],
        out_specs=pl.BlockSpec((1, 1, 1, 2 * _C), lambda b, c: (b, c, 0, 0)),
        out_shape=jax.ShapeDtypeStruct((B, _NCHUNK, 1, 2 * _C), jnp.float32),
        compiler_params=pltpu.CompilerParams(
            vmem_limit_bytes=110 * 1024 * 1024,
        ),
    )(P_batch, Q_batch, projc)
    wpp_full = wpp[:, :, 0, :_C].transpose(0, 1, 2).reshape(B, _NCHUNK * _C)
    swd = jnp.sqrt(jnp.mean(wpp_full[:, :_L], axis=1))
    return jnp.sum(swd) / B


# speedup vs baseline: 1.0493x; 1.0493x over previous
import jax
import jax.numpy as jnp
from jax import lax
from jax.experimental import pallas as pl
from jax.experimental.pallas import tpu as pltpu

_N = 8192
_D = 128
_L = 100
_C = 64
_NCHUNK = 2

def _a_stage(x, k, j, W):
    g = _N // (2 * j)
    xr = x.reshape(g, 2, j, W)
    a = xr[:, 0]
    b = xr[:, 1]
    giota = lax.broadcasted_iota(jnp.int32, (g, 1, 1), 0) * (2 * j)
    gasc = (giota & k) == 0
    lo = jnp.minimum(a, b)
    hi = jnp.maximum(a, b)
    first = jnp.where(gasc, lo, hi)
    second = jnp.where(gasc, hi, lo)
    return jnp.stack([first, second], axis=1).reshape(_N, W)


def _b_stage(x, k, j, W):
    rows = _N // 8
    if k >= 8:
        riota = lax.broadcasted_iota(jnp.int32, (rows, 1), 0)
        rasc = (riota & (k // 8)) == 0
    pieces = []
    for t in range(8):
        a = x[:, t * W:(t + 1) * W]
        b = x[:, (t ^ j) * W:((t ^ j) + 1) * W]
        lo = jnp.minimum(a, b)
        hi = jnp.maximum(a, b)
        bc = (t & j) == 0
        if k < 8:
            take_lo = ((t & k) == 0) == bc
            pieces.append(lo if take_lo else hi)
        else:
            pieces.append(jnp.where(rasc == bc, lo, hi))
    return jnp.concatenate(pieces, axis=1)


def _sort_cols(x):
    W = x.shape[1]
    rows = _N // 8
    xb = x.reshape(rows, 8 * W)
    for k in (2, 4, 8):
        j = k // 2
        while j >= 1:
            xb = _b_stage(xb, k, j, W)
            j //= 2
    k = 16
    while k <= _N:
        xa = xb.reshape(_N, W)
        j = k // 2
        while j >= 8:
            xa = _a_stage(xa, k, j, W)
            j //= 2
        xb = xa.reshape(rows, 8 * W)
        for j in (4, 2, 1):
            xb = _b_stage(xb, k, j, W)
        k *= 2
    return xb


def _swd_kernel(p_ref, q_ref, proj_ref, out_ref):
    P = p_ref[0]
    Q = q_ref[0]
    proj = proj_ref[0]
    Xp = jnp.dot(P, proj, preferred_element_type=jnp.float32)
    Yp = jnp.dot(Q, proj, preferred_element_type=jnp.float32)
    xb = _sort_cols(jnp.concatenate([Xp, Yp], axis=1))
    W = 2 * _C
    acc = jnp.zeros((_N // 8, _C), jnp.float32)
    for t in range(8):
        xt = xb[:, t * W:(t + 1) * W]
        d = xt[:, :_C] - xt[:, _C:]
        acc = acc + d * d
    m = jnp.sum(acc, axis=0, keepdims=True) * (1.0 / _N)
    out_ref[0, 0] = jnp.concatenate([m, jnp.zeros((1, _C), jnp.float32)], axis=1)


def kernel(P_batch, Q_batch, projections):
    B = P_batch.shape[0]
    projp = jnp.zeros((_D, _NCHUNK * _C), jnp.float32).at[:, :_L].set(projections)
    projc = projp.reshape(_D, _NCHUNK, _C).transpose(1, 0, 2)
    wpp = pl.pallas_call(
        _swd_kernel,
        grid=(B, _NCHUNK),
        in_specs=[
            pl.BlockSpec((1, _N, _D), lambda b, c: (b, 0, 0)),
            pl.BlockSpec((1, _N, _D), lambda b, c: (b, 0, 0)),
            pl.BlockSpec((1, _D, _C), lambda b, c: (c, 0, 0)),
        ],
        out_specs=pl.BlockSpec((1, 1, 1, 2 * _C), lambda b, c: (b, c, 0, 0)),
        out_shape=jax.ShapeDtypeStruct((B, _NCHUNK, 1, 2 * _C), jnp.float32),
        compiler_params=pltpu.CompilerParams(
            vmem_limit_bytes=110 * 1024 * 1024,
        ),
    )(P_batch, Q_batch, projc)
    wpp_full = wpp[:, :, 0, :_C].transpose(0, 1, 2).reshape(B, _NCHUNK * _C)
    swd = jnp.sqrt(jnp.mean(wpp_full[:, :_L], axis=1))
    return jnp.sum(swd) / B


# speedup vs baseline: 1.0586x; 1.0088x over previous
import jax
import jax.numpy as jnp
from jax import lax
from jax.experimental import pallas as pl
from jax.experimental.pallas import tpu as pltpu

_N = 8192
_D = 128
_L = 100
_C = 64
_NCHUNK = 2

def _a_stage(x, k, j, W):
    g = _N // (2 * j)
    xr = x.reshape(g, 2, j, W)
    a = xr[:, 0]
    b = xr[:, 1]
    giota = lax.broadcasted_iota(jnp.int32, (g, 1, 1), 0) * (2 * j)
    gasc = (giota & k) == 0
    lo = jnp.minimum(a, b)
    hi = jnp.maximum(a, b)
    first = jnp.where(gasc, lo, hi)
    second = jnp.where(gasc, hi, lo)
    return jnp.stack([first, second], axis=1).reshape(_N, W)


def _b_stage(x, k, j, W):
    rows = _N // 8
    if k >= 8:
        riota = lax.broadcasted_iota(jnp.int32, (rows, 1), 0)
        rasc = (riota & (k // 8)) == 0
    pieces = []
    for t in range(8):
        a = x[:, t * W:(t + 1) * W]
        b = x[:, (t ^ j) * W:((t ^ j) + 1) * W]
        lo = jnp.minimum(a, b)
        hi = jnp.maximum(a, b)
        bc = (t & j) == 0
        if k < 8:
            take_lo = ((t & k) == 0) == bc
            pieces.append(lo if take_lo else hi)
        else:
            pieces.append(jnp.where(rasc == bc, lo, hi))
    return jnp.concatenate(pieces, axis=1)


def _sort_cols(x):
    W = x.shape[1]
    rows = _N // 8
    xb = x.reshape(rows, 8 * W)
    for k in (2, 4, 8):
        j = k // 2
        while j >= 1:
            xb = _b_stage(xb, k, j, W)
            j //= 2
    k = 16
    while k <= _N:
        xa = xb.reshape(_N, W)
        j = k // 2
        while j >= 8:
            xa = _a_stage(xa, k, j, W)
            j //= 2
        xb = xa.reshape(rows, 8 * W)
        for j in (4, 2, 1):
            xb = _b_stage(xb, k, j, W)
        k *= 2
    return xb.reshape(_N, W)


def _swd_kernel(p_ref, q_ref, proj_ref, out_ref):
    P = p_ref[0]
    Q = q_ref[0]
    proj = proj_ref[0]
    Xp = jnp.dot(P, proj, preferred_element_type=jnp.float32)
    Yp = jnp.dot(Q, proj, preferred_element_type=jnp.float32)
    x = _sort_cols(jnp.concatenate([Xp, Yp], axis=1))
    d = x[:, :_C] - x[:, _C:]
    m = jnp.mean(d * d, axis=0, keepdims=True)
    out_ref[0, 0] = jnp.concatenate([m, jnp.zeros((1, _C), jnp.float32)], axis=1)


def kernel(P_batch, Q_batch, projections):
    B = P_batch.shape[0]
    projp = jnp.zeros((_D, _NCHUNK * _C), jnp.float32).at[:, :_L].set(projections)
    projc = projp.reshape(_D, _NCHUNK, _C).transpose(1, 0, 2)
    wpp = pl.pallas_call(
        _swd_kernel,
        grid=(B, _NCHUNK),
        in_specs=[
            pl.BlockSpec((1, _N, _D), lambda b, c: (b, 0, 0)),
            pl.BlockSpec((1, _N, _D), lambda b, c: (b, 0, 0)),
            pl.BlockSpec((1, _D, _C), lambda b, c: (c, 0, 0)),
        ],
        out_specs=pl.BlockSpec((1, 1, 1, 2 * _C), lambda b, c: (b, c, 0, 0)),
        out_shape=jax.ShapeDtypeStruct((B, _NCHUNK, 1, 2 * _C), jnp.float32),
        compiler_params=pltpu.CompilerParams(
            vmem_limit_bytes=110 * 1024 * 1024,
        ),
    )(P_batch, Q_batch, projc)
    wpp_full = wpp[:, :, 0, :_C].reshape(B, _NCHUNK * _C)
    swd = jnp.sqrt(jnp.mean(wpp_full[:, :_L], axis=1))
    return jnp.sum(swd) / B
